# baseline (speedup 1.0000x reference)
import functools

import jax
import jax.numpy as jnp
from jax import lax
from jax.experimental import pallas as pl
from jax.experimental.pallas import tpu as pltpu
from jax.experimental.pallas import tpu_sc as plsc

_NBUF = 4
_C = 128


def _make_sc_gather(V, D, N):
    info = plsc.get_sparse_core_info()
    NC, NS = info.num_cores, info.num_subcores
    NW = NC * NS

    per_w = N // NW
    n_chunks = per_w // _C
    assert N % NW == 0 and per_w % _C == 0 and n_chunks % _NBUF == 0

    mesh = plsc.VectorSubcoreMesh(core_axis_name="c", subcore_axis_name="s")

    @functools.partial(
        pl.kernel,
        mesh=mesh,
        out_type=jax.ShapeDtypeStruct((N, D), jnp.float32),
        scratch_types=[pltpu.VMEM((per_w,), jnp.int32)]
        + [pltpu.VMEM((_NBUF, _C, D), jnp.float32)]
        + [pltpu.SemaphoreType.DMA] * (2 * _NBUF),
        compiler_params=pltpu.CompilerParams(use_tc_tiling_on_sc=False),
    )
    def grab(idx_hbm, table_hbm, out_hbm, idx_v, rows_v, *sems):
        gsem = sems[:_NBUF]
        ssem = sems[_NBUF:]
        wid = lax.axis_index("s") * NC + lax.axis_index("c")
        base = wid * per_w

        pltpu.sync_copy(idx_hbm.at[pl.ds(base, per_w)], idx_v)

        def fire_gather(c, b):
            pltpu.async_copy(
                table_hbm.at[idx_v.at[pl.ds(c * _C, _C)]],
                rows_v.at[b],
                gsem[b],
            )

        def wait_gather(c, b):
            pltpu.make_async_copy(
                table_hbm.at[idx_v.at[pl.ds(c * _C, _C)]], rows_v.at[b], gsem[b]
            ).wait()

        def fire_store(c, b):
            pltpu.async_copy(
                rows_v.at[b], out_hbm.at[pl.ds(base + c * _C, _C)], ssem[b]
            )

        def wait_store(b):
            pltpu.make_async_copy(
                rows_v.at[b], out_hbm.at[pl.ds(0, _C)], ssem[b]
            ).wait()

        for b in range(_NBUF - 1):
            fire_gather(b, b)

        def tick(c0, _):
            for b in range(_NBUF):
                c = c0 + b
                wait_gather(c, b)
                fire_store(c, b)
                t = (b + _NBUF - 1) % _NBUF

                @pl.when(c + _NBUF - 1 < n_chunks)
                def _():
                    @pl.when(c >= 1)
                    def _():
                        wait_store(t)

                    fire_gather(c + _NBUF - 1, t)

            return ()

        lax.fori_loop(0, n_chunks // _NBUF, lambda i, _: tick(i * _NBUF, _), ())

        for b in range(_NBUF):
            wait_store(b)

    return grab


def kernel(input_seq, table):
    B, H = input_seq.shape
    V, D = table.shape
    N = B * H
    grab = _make_sc_gather(V, D, N)
    flat_idx = input_seq.astype(jnp.int32).reshape(N)
    out = grab(flat_idx, table)
    return out.reshape(B, H, D)


# speedup vs baseline: 1.0118x; 1.0118x over previous
import functools

import jax
import jax.numpy as jnp
from jax import lax
from jax.experimental import pallas as pl
from jax.experimental.pallas import tpu as pltpu
from jax.experimental.pallas import tpu_sc as plsc

_C = 128
_G = 8


def _make_sc_gather(V, D, N):
    info = plsc.get_sparse_core_info()
    NC, NS = info.num_cores, info.num_subcores
    NW = NC * NS

    nch = N // _C
    ch_per_w = nch // NW
    ngrp = ch_per_w // _G
    assert N % _C == 0 and nch % NW == 0 and ch_per_w % _G == 0
    assert ngrp % 2 == 0 and ngrp >= 4

    mesh = plsc.VectorSubcoreMesh(core_axis_name="c", subcore_axis_name="s")

    @functools.partial(
        pl.kernel,
        mesh=mesh,
        out_type=jax.ShapeDtypeStruct((nch, _C, D), jnp.float32),
        scratch_types=[pltpu.VMEM((2, _G, _C), jnp.int32)]
        + [pltpu.VMEM((2, _G, _C, D), jnp.float32)]
        + [pltpu.SemaphoreType.DMA] * 6,
        compiler_params=pltpu.CompilerParams(use_tc_tiling_on_sc=False),
    )
    def grab(idx_hbm, table_hbm, out_hbm, idx_v, rows_v, *sems):
        isem = sems[0:2]
        gsem = sems[2:4]
        ssem = sems[4:6]
        wid = lax.axis_index("s") * NC + lax.axis_index("c")
        wbase = wid * ch_per_w

        def fire_idx(g, s):
            pltpu.async_copy(
                idx_hbm.at[pl.ds(wbase + g * _G, _G)], idx_v.at[s], isem[s]
            )

        def wait_idx(s):
            pltpu.make_async_copy(
                idx_hbm.at[pl.ds(0, _G)], idx_v.at[s], isem[s]
            ).wait()

        def fire_gather(s):
            for j in range(_G):
                pltpu.async_copy(
                    table_hbm.at[idx_v.at[s].at[j]], rows_v.at[s].at[j], gsem[s]
                )

        def wait_gather(s):
            for j in range(_G):
                pltpu.make_async_copy(
                    table_hbm.at[idx_v.at[s].at[j]], rows_v.at[s].at[j], gsem[s]
                ).wait()

        def fire_store(g, s):
            pltpu.async_copy(
                rows_v.at[s], out_hbm.at[pl.ds(wbase + g * _G, _G)], ssem[s]
            )

        def wait_store(s):
            pltpu.make_async_copy(
                rows_v.at[s], out_hbm.at[pl.ds(0, _G)], ssem[s]
            ).wait()

        fire_idx(0, 0)
        fire_idx(1, 1)
        wait_idx(0)
        fire_gather(0)

        def tick(g, s):
            wait_gather(s)
            fire_store(g, s)

            @pl.when(g + 2 < ngrp)
            def _():
                fire_idx(g + 2, s)

            @pl.when(g + 1 < ngrp)
            def _():
                wait_idx(1 - s)

                @pl.when(g >= 1)
                def _():
                    wait_store(1 - s)

                fire_gather(1 - s)

        def body(i, _):
            tick(2 * i, 0)
            tick(2 * i + 1, 1)
            return ()

        lax.fori_loop(0, ngrp // 2, body, ())

        wait_store(0)
        wait_store(1)

    return grab


def kernel(input_seq, table):
    B, H = input_seq.shape
    V, D = table.shape
    N = B * H
    grab = _make_sc_gather(V, D, N)
    idx2d = input_seq.astype(jnp.int32).reshape(N // _C, _C)
    out = grab(idx2d, table)
    return out.reshape(B, H, D)
